# Initial kernel scaffold; baseline (speedup 1.0000x reference)
import functools

import jax
import jax.numpy as jnp
from jax import lax
from jax.experimental import pallas as pl
from jax.experimental.pallas import tpu as pltpu
from jax.experimental.pallas import tpu_sc as plsc

N = 10000
E = 320000
D = 128
H = 64
G = 256

NC = 2
NS = 16
NW = NC * NS
CW = 128

ECHUNKS = 79
EPAD = NW * ECHUNKS * CW
NB = N + 16
ROWS_PER_TILE = NB // NS

BCHUNKS = 3
NPAD = NW * BCHUNKS * CW
PB = G + 16

_f32 = jnp.float32
_mesh = plsc.VectorSubcoreMesh(core_axis_name="c", subcore_axis_name="s")


def _hist(cols3d, batch3d, ones16, zeros16):
    out_deg = jax.ShapeDtypeStruct((NC, NB, 16), _f32)
    out_cnt = jax.ShapeDtypeStruct((NC, PB, 16), _f32)

    @functools.partial(
        pl.kernel,
        out_type=(out_deg, out_cnt),
        mesh=_mesh,
        scratch_types=[
            pltpu.VMEM_SHARED((NB, 16), _f32),
            pltpu.VMEM_SHARED((PB, 16), _f32),
            pltpu.VMEM((ECHUNKS, CW), jnp.int32),
            pltpu.VMEM((BCHUNKS, CW), jnp.int32),
            pltpu.VMEM((CW, 16), _f32),
        ],
    )
    def k(cols_hbm, batch_hbm, ones_hbm, z_hbm, deg_hbm, cnt_hbm,
          dacc, cacc, cidx, bidx, ones_v):
        c = lax.axis_index("c")
        s = lax.axis_index("s")
        wid = s * NC + c
        pltpu.sync_copy(z_hbm.at[pl.ds(0, ROWS_PER_TILE)],
                        dacc.at[pl.ds(s * ROWS_PER_TILE, ROWS_PER_TILE)])
        pltpu.sync_copy(z_hbm.at[pl.ds(0, PB // NS)],
                        cacc.at[pl.ds(s * (PB // NS), PB // NS)])
        pltpu.sync_copy(cols_hbm.at[wid], cidx)
        pltpu.sync_copy(batch_hbm.at[wid], bidx)
        pltpu.sync_copy(ones_hbm, ones_v)
        plsc.subcore_barrier()

        @pl.loop(0, ECHUNKS)
        def _(j):
            pltpu.sync_copy(ones_v, dacc.at[cidx.at[j]], add=True)

        for j in range(BCHUNKS):
            pltpu.sync_copy(ones_v, cacc.at[bidx.at[j]], add=True)
        plsc.subcore_barrier()
        pltpu.sync_copy(dacc.at[pl.ds(s * ROWS_PER_TILE, ROWS_PER_TILE)],
                        deg_hbm.at[c, pl.ds(s * ROWS_PER_TILE, ROWS_PER_TILE)])
        pltpu.sync_copy(cacc.at[pl.ds(s * (PB // NS), PB // NS)],
                        cnt_hbm.at[c, pl.ds(s * (PB // NS), PB // NS)])

    return k(cols3d, batch3d, ones16, zeros16)


def _aggregate(g, rows3d, cols3d, zeros64):
    @functools.partial(
        pl.kernel,
        out_type=jax.ShapeDtypeStruct((NC, NB, H), _f32),
        mesh=_mesh,
        scratch_types=[
            pltpu.VMEM_SHARED((NB, H), _f32),
            pltpu.VMEM((ECHUNKS, CW), jnp.int32),
            pltpu.VMEM((ECHUNKS, CW), jnp.int32),
            pltpu.VMEM((CW, H), _f32),
        ],
    )
    def k(g_hbm, rows_hbm, cols_hbm, z_hbm, out_hbm, acc, ridx, cidx, gbuf):
        c = lax.axis_index("c")
        s = lax.axis_index("s")
        wid = s * NC + c
        pltpu.sync_copy(z_hbm.at[pl.ds(0, ROWS_PER_TILE)],
                        acc.at[pl.ds(s * ROWS_PER_TILE, ROWS_PER_TILE)])
        pltpu.sync_copy(rows_hbm.at[wid], ridx)
        pltpu.sync_copy(cols_hbm.at[wid], cidx)
        plsc.subcore_barrier()

        @pl.loop(0, ECHUNKS)
        def _(j):
            pltpu.sync_copy(g_hbm.at[ridx.at[j]], gbuf)
            pltpu.sync_copy(gbuf, acc.at[cidx.at[j]], add=True)

        plsc.subcore_barrier()
        pltpu.sync_copy(acc.at[pl.ds(s * ROWS_PER_TILE, ROWS_PER_TILE)],
                        out_hbm.at[c, pl.ds(s * ROWS_PER_TILE, ROWS_PER_TILE)])

    return k(g, rows3d, cols3d, zeros64)


def _pool(h3p, batch3d, zeros64):
    @functools.partial(
        pl.kernel,
        out_type=jax.ShapeDtypeStruct((NC, G, H), _f32),
        mesh=_mesh,
        scratch_types=[
            pltpu.VMEM_SHARED((PB, H), _f32),
            pltpu.VMEM((BCHUNKS * CW, H), _f32),
            pltpu.VMEM((BCHUNKS, CW), jnp.int32),
        ],
    )
    def k(h_hbm, batch_hbm, z_hbm, out_hbm, acc, vbuf, bidx):
        c = lax.axis_index("c")
        s = lax.axis_index("s")
        wid = s * NC + c
        pltpu.sync_copy(z_hbm.at[pl.ds(0, PB // NS)],
                        acc.at[pl.ds(s * (PB // NS), PB // NS)])
        pltpu.sync_copy(h_hbm.at[pl.ds(wid * (BCHUNKS * CW), BCHUNKS * CW)],
                        vbuf)
        pltpu.sync_copy(batch_hbm.at[wid], bidx)
        plsc.subcore_barrier()
        for j in range(BCHUNKS):
            pltpu.sync_copy(vbuf.at[pl.ds(j * CW, CW)],
                            acc.at[bidx.at[j]], add=True)
        plsc.subcore_barrier()
        pltpu.sync_copy(acc.at[pl.ds(s * (G // NS), G // NS)],
                        out_hbm.at[c, pl.ds(s * (G // NS), G // NS)])

    return k(h3p, batch3d, zeros64)


def _dis_from(deg_ref):
    deg = deg_ref[0, :N, :1] + deg_ref[1, :N, :1] + 1.0
    return 1.0 / jnp.sqrt(deg)


def _l1_body(x_ref, w_ref, deg_ref, o_ref):
    dis = _dis_from(deg_ref)
    g = jnp.dot(x_ref[...], w_ref[...], preferred_element_type=_f32)
    o_ref[...] = g * dis


def _mid_body(acc_ref, g_ref, deg_ref, b_ref, w_ref, o_ref):
    dis = _dis_from(deg_ref)
    tot = acc_ref[0, :N, :] + acc_ref[1, :N, :] + g_ref[...]
    h = jnp.maximum(tot * dis + b_ref[...], 0.0)
    o_ref[...] = jnp.dot(h, w_ref[...], preferred_element_type=_f32) * dis


def _fin_body(acc_ref, g_ref, deg_ref, b_ref, o_ref):
    dis = _dis_from(deg_ref)
    tot = acc_ref[0, :N, :] + acc_ref[1, :N, :] + g_ref[...]
    h = jnp.maximum(tot * dis + b_ref[...], 0.0)
    o_ref[:N, :] = h
    o_ref[N:, :] = jnp.zeros((NPAD - N, H), _f32)


def _mlp_body(accp_ref, cnt_ref, w1_ref, b1_ref, w2_ref, b2_ref, o_ref):
    sums = accp_ref[0] + accp_ref[1]
    cnts = cnt_ref[0, :G, :1] + cnt_ref[1, :G, :1]
    pooled = sums / jnp.maximum(cnts, 1.0)
    r = jnp.maximum(
        jnp.dot(pooled, w1_ref[...], preferred_element_type=_f32)
        + b1_ref[...], 0.0)
    o_ref[...] = (jnp.dot(r, w2_ref[...], preferred_element_type=_f32)
                  + b2_ref[...])


def _tc_call(body, out_shape, *args):
    return pl.pallas_call(
        body, out_shape=jax.ShapeDtypeStruct(out_shape, _f32))(*args)


def kernel(x, edge_index, batch, W1, b1, W2, b2, W3, b3,
           fc1_W, fc1_b, fc2_W, fc2_b):
    row = edge_index[0]
    col = edge_index[1]
    rows3d = jnp.pad(row, (0, EPAD - E)).reshape(NW, ECHUNKS, CW)
    cols3d = jnp.pad(col, (0, EPAD - E),
                     constant_values=N).reshape(NW, ECHUNKS, CW)
    batch3d = jnp.pad(batch, (0, NPAD - N),
                      constant_values=G).reshape(NW, BCHUNKS, CW)
    zeros64 = jnp.zeros((ROWS_PER_TILE, H), _f32)
    zeros16 = jnp.zeros((ROWS_PER_TILE, 16), _f32)
    ones16 = jnp.ones((CW, 16), _f32)

    deg, cnt = _hist(cols3d, batch3d, ones16, zeros16)

    g1 = _tc_call(_l1_body, (N, H), x, W1, deg)
    acc1 = _aggregate(g1, rows3d, cols3d, zeros64)
    g2 = _tc_call(_mid_body, (N, H), acc1, g1, deg, b1.reshape(1, H), W2)
    acc2 = _aggregate(g2, rows3d, cols3d, zeros64)
    g3 = _tc_call(_mid_body, (N, H), acc2, g2, deg, b2.reshape(1, H), W3)
    acc3 = _aggregate(g3, rows3d, cols3d, zeros64)
    h3p = _tc_call(_fin_body, (NPAD, H), acc3, g3, deg, b3.reshape(1, H))

    accp = _pool(h3p, batch3d, zeros64)
    out = _tc_call(_mlp_body, (G, 1), accp, cnt, fc1_W,
                   fc1_b.reshape(1, 10), fc2_W, fc2_b.reshape(1, 1))
    return out


# speedup vs baseline: 16.7802x; 16.7802x over previous
import functools

import jax
import jax.numpy as jnp
from jax import lax
from jax.experimental import pallas as pl
from jax.experimental.pallas import tpu as pltpu
from jax.experimental.pallas import tpu_sc as plsc

N = 10000
E = 320000
D = 128
H = 64
G = 256

NC = 2
NS = 16
NW = NC * NS
CW = 128

ECHUNKS = 79
EPAD = NW * ECHUNKS * CW
NB = N + 112
ROWS_PER_TILE = NB // NS

BCHUNKS = 3
NPAD = NW * BCHUNKS * CW
PB = G + 128

_f32 = jnp.float32


def _mesh():
    return plsc.VectorSubcoreMesh(core_axis_name="c", subcore_axis_name="s",
                                  num_cores=NC, num_subcores=NS)


def _hist(cols3d, batch3d, ones16, zeros16):
    out_deg = jax.ShapeDtypeStruct((NC, NB, 16), _f32)
    out_cnt = jax.ShapeDtypeStruct((NC, PB, 16), _f32)

    @functools.partial(
        pl.kernel,
        compiler_params=pltpu.CompilerParams(use_tc_tiling_on_sc=False),
        out_type=(out_deg, out_cnt),
        mesh=_mesh(),
        scratch_types=[
            pltpu.VMEM_SHARED((NB, 16), _f32),
            pltpu.VMEM_SHARED((PB, 16), _f32),
            pltpu.VMEM((ECHUNKS, CW), jnp.int32),
            pltpu.VMEM((BCHUNKS, CW), jnp.int32),
            pltpu.VMEM((CW, 16), _f32),
        ],
    )
    def k(cols_hbm, batch_hbm, ones_hbm, z_hbm, deg_hbm, cnt_hbm,
          dacc, cacc, cidx, bidx, ones_v):
        c = lax.axis_index("c")
        s = lax.axis_index("s")
        wid = s * NC + c
        pltpu.sync_copy(z_hbm.at[pl.ds(0, ROWS_PER_TILE)],
                        dacc.at[pl.ds(s * ROWS_PER_TILE, ROWS_PER_TILE)])
        pltpu.sync_copy(z_hbm.at[pl.ds(0, PB // NS)],
                        cacc.at[pl.ds(s * (PB // NS), PB // NS)])
        pltpu.sync_copy(cols_hbm.at[wid], cidx)
        pltpu.sync_copy(batch_hbm.at[wid], bidx)
        pltpu.sync_copy(ones_hbm, ones_v)
        plsc.subcore_barrier()

        @pl.loop(0, ECHUNKS)
        def _(j):
            pltpu.sync_copy(ones_v, dacc.at[cidx.at[j]], add=True)

        for j in range(BCHUNKS):
            pltpu.sync_copy(ones_v, cacc.at[bidx.at[j]], add=True)
        plsc.subcore_barrier()
        pltpu.sync_copy(dacc.at[pl.ds(s * ROWS_PER_TILE, ROWS_PER_TILE)],
                        deg_hbm.at[c, pl.ds(s * ROWS_PER_TILE, ROWS_PER_TILE)])
        pltpu.sync_copy(cacc.at[pl.ds(s * (PB // NS), PB // NS)],
                        cnt_hbm.at[c, pl.ds(s * (PB // NS), PB // NS)])

    return k(cols3d, batch3d, ones16, zeros16)


def _aggregate(g, rows3d, cols3d, zeros64):
    @functools.partial(
        pl.kernel,
        compiler_params=pltpu.CompilerParams(use_tc_tiling_on_sc=False),
        out_type=jax.ShapeDtypeStruct((NC, NB, H), _f32),
        mesh=_mesh(),
        scratch_types=[
            pltpu.VMEM_SHARED((NB, H), _f32),
            pltpu.VMEM((ECHUNKS, CW), jnp.int32),
            pltpu.VMEM((ECHUNKS, CW), jnp.int32),
            pltpu.VMEM((CW, H), _f32),
        ],
    )
    def k(g_hbm, rows_hbm, cols_hbm, z_hbm, out_hbm, acc, ridx, cidx, gbuf):
        c = lax.axis_index("c")
        s = lax.axis_index("s")
        wid = s * NC + c
        pltpu.sync_copy(z_hbm.at[pl.ds(0, ROWS_PER_TILE)],
                        acc.at[pl.ds(s * ROWS_PER_TILE, ROWS_PER_TILE)])
        pltpu.sync_copy(rows_hbm.at[wid], ridx)
        pltpu.sync_copy(cols_hbm.at[wid], cidx)
        plsc.subcore_barrier()

        @pl.loop(0, ECHUNKS)
        def _(j):
            pltpu.sync_copy(g_hbm.at[ridx.at[j]], gbuf)
            pltpu.sync_copy(gbuf, acc.at[cidx.at[j]], add=True)

        plsc.subcore_barrier()
        pltpu.sync_copy(acc.at[pl.ds(s * ROWS_PER_TILE, ROWS_PER_TILE)],
                        out_hbm.at[c, pl.ds(s * ROWS_PER_TILE, ROWS_PER_TILE)])

    return k(g, rows3d, cols3d, zeros64)


def _pool(h3p, batch3d, zeros64):
    @functools.partial(
        pl.kernel,
        compiler_params=pltpu.CompilerParams(use_tc_tiling_on_sc=False),
        out_type=jax.ShapeDtypeStruct((NC, G, H), _f32),
        mesh=_mesh(),
        scratch_types=[
            pltpu.VMEM_SHARED((PB, H), _f32),
            pltpu.VMEM((BCHUNKS * CW, H), _f32),
            pltpu.VMEM((BCHUNKS, CW), jnp.int32),
        ],
    )
    def k(h_hbm, batch_hbm, z_hbm, out_hbm, acc, vbuf, bidx):
        c = lax.axis_index("c")
        s = lax.axis_index("s")
        wid = s * NC + c
        pltpu.sync_copy(z_hbm.at[pl.ds(0, PB // NS)],
                        acc.at[pl.ds(s * (PB // NS), PB // NS)])
        pltpu.sync_copy(h_hbm.at[pl.ds(wid * (BCHUNKS * CW), BCHUNKS * CW)],
                        vbuf)
        pltpu.sync_copy(batch_hbm.at[wid], bidx)
        plsc.subcore_barrier()
        for j in range(BCHUNKS):
            pltpu.sync_copy(vbuf.at[pl.ds(j * CW, CW)],
                            acc.at[bidx.at[j]], add=True)
        plsc.subcore_barrier()
        pltpu.sync_copy(acc.at[pl.ds(s * (G // NS), G // NS)],
                        out_hbm.at[c, pl.ds(s * (G // NS), G // NS)])

    return k(h3p, batch3d, zeros64)


def _dis_from(deg_ref):
    deg = deg_ref[0, :N, :1] + deg_ref[1, :N, :1] + 1.0
    return 1.0 / jnp.sqrt(deg)


def _l1_body(x_ref, w_ref, deg_ref, o_ref):
    dis = _dis_from(deg_ref)
    g = jnp.dot(x_ref[...], w_ref[...], preferred_element_type=_f32)
    o_ref[...] = g * dis


def _mid_body(acc_ref, g_ref, deg_ref, b_ref, w_ref, o_ref):
    dis = _dis_from(deg_ref)
    tot = acc_ref[0, :N, :] + acc_ref[1, :N, :] + g_ref[...]
    h = jnp.maximum(tot * dis + b_ref[...], 0.0)
    o_ref[...] = jnp.dot(h, w_ref[...], preferred_element_type=_f32) * dis


def _fin_body(acc_ref, g_ref, deg_ref, b_ref, o_ref):
    dis = _dis_from(deg_ref)
    tot = acc_ref[0, :N, :] + acc_ref[1, :N, :] + g_ref[...]
    h = jnp.maximum(tot * dis + b_ref[...], 0.0)
    o_ref[:N, :] = h
    o_ref[N:, :] = jnp.zeros((NPAD - N, H), _f32)


def _mlp_body(accp_ref, cnt_ref, w1_ref, b1_ref, w2_ref, b2_ref, o_ref):
    sums = accp_ref[0] + accp_ref[1]
    cnts = cnt_ref[0, :G, :1] + cnt_ref[1, :G, :1]
    pooled = sums / jnp.maximum(cnts, 1.0)
    r = jnp.maximum(
        jnp.dot(pooled, w1_ref[...], preferred_element_type=_f32)
        + b1_ref[...], 0.0)
    o_ref[...] = (jnp.dot(r, w2_ref[...], preferred_element_type=_f32)
                  + b2_ref[...])


def _tc_call(body, out_shape, *args):
    return pl.pallas_call(
        body, out_shape=jax.ShapeDtypeStruct(out_shape, _f32))(*args)


def kernel(x, edge_index, batch, W1, b1, W2, b2, W3, b3,
           fc1_W, fc1_b, fc2_W, fc2_b):
    row = edge_index[0]
    col = edge_index[1]
    rows3d = jnp.pad(row, (0, EPAD - E)).reshape(NW, ECHUNKS, CW)
    cols3d = jnp.pad(col, (0, EPAD - E),
                     constant_values=N).reshape(NW, ECHUNKS, CW)
    batch3d = jnp.pad(batch, (0, NPAD - N),
                      constant_values=G).reshape(NW, BCHUNKS, CW)
    zeros64 = jnp.zeros((ROWS_PER_TILE, H), _f32)
    zeros16 = jnp.zeros((ROWS_PER_TILE, 16), _f32)
    ones16 = jnp.ones((CW, 16), _f32)

    deg, cnt = _hist(cols3d, batch3d, ones16, zeros16)

    g1 = _tc_call(_l1_body, (N, H), x, W1, deg)
    acc1 = _aggregate(g1, rows3d, cols3d, zeros64)
    g2 = _tc_call(_mid_body, (N, H), acc1, g1, deg, b1.reshape(1, H), W2)
    acc2 = _aggregate(g2, rows3d, cols3d, zeros64)
    g3 = _tc_call(_mid_body, (N, H), acc2, g2, deg, b2.reshape(1, H), W3)
    acc3 = _aggregate(g3, rows3d, cols3d, zeros64)
    h3p = _tc_call(_fin_body, (NPAD, H), acc3, g3, deg, b3.reshape(1, H))

    accp = _pool(h3p, batch3d, zeros64)
    out = _tc_call(_mlp_body, (G, 1), accp, cnt, fc1_W,
                   fc1_b.reshape(1, 10), fc2_W, fc2_b.reshape(1, 1))
    return out
